# baseline (speedup 1.0000x reference)
import functools

import jax
import jax.numpy as jnp
from jax import lax
from jax.experimental import pallas as pl
from jax.experimental.pallas import tpu as pltpu
from jax.experimental.pallas import tpu_sc as plsc

NBINS = 5
H = 480
W = 640
B = 8
N = 1_000_000
HWSZ = H * W
G = NBINS * HWSZ

NSUB = 16
NCORE = 2
BATCH_PER_CORE = B // NCORE
CHUNK = 1600
NCHUNKS = N // CHUNK
VECS = CHUNK // 16
ROWS = 2 * CHUNK // 128
STRIPE = G // NSUB
ZCHUNK = 12_000


def _make_sc_kernel():
    mesh = plsc.VectorSubcoreMesh(core_axis_name="c", subcore_axis_name="s")

    @functools.partial(
        pl.kernel,
        out_type=jax.ShapeDtypeStruct((B, G), jnp.float32),
        mesh=mesh,
        scratch_types=[
            pltpu.VMEM((CHUNK * 4,), jnp.float32),
            pltpu.VMEM((ROWS, 128), jnp.int32),
            pltpu.VMEM((ROWS, 128), jnp.float32),
            pltpu.VMEM((ZCHUNK,), jnp.float32),
            pltpu.VMEM((16,), jnp.float32),
            pltpu.VMEM_SHARED((G,), jnp.float32),
        ],
        compiler_params=pltpu.CompilerParams(needs_layout_passes=False),
    )
    def voxel_sc(ev_hbm, stamps_hbm, out_hbm, ev_v, idx_v, val_v, zbuf,
                 stamps_v, grid):
        c = lax.axis_index("c")
        s = lax.axis_index("s")

        def zb(i, carry):
            zbuf[pl.ds(i * 16, 16)] = jnp.zeros((16,), jnp.float32)
            return carry
        lax.fori_loop(0, ZCHUNK // 16, zb, 0)
        pltpu.sync_copy(stamps_hbm, stamps_v)

        lanes = lax.broadcasted_iota(jnp.int32, (16,), 0)

        def batch_body(bi, carry):
            b = c * BATCH_PER_CORE + bi
            def zg(j, cc):
                pltpu.sync_copy(
                    zbuf, grid.at[pl.ds(s * STRIPE + j * ZCHUNK, ZCHUNK)])
                return cc
            lax.fori_loop(0, STRIPE // ZCHUNK, zg, 0)
            plsc.subcore_barrier()

            b2 = lanes * 0 + b * 2
            first = plsc.load_gather(stamps_v, [b2])
            last = plsc.load_gather(stamps_v, [b2 + 1])
            d0 = last - first
            dT = jnp.where(d0 == 0.0, jnp.float32(1.0), d0)
            first4 = first * jnp.float32(4.0)

            def compute_vec(v):
                rows4 = v * 64 + lanes * 4
                t = plsc.load_gather(ev_v, [rows4])
                x = plsc.load_gather(ev_v, [rows4 + 1])
                y = plsc.load_gather(ev_v, [rows4 + 2])
                p = plsc.load_gather(ev_v, [rows4 + 3])
                ts = (t * jnp.float32(4.0) - first4) / dT
                trunc = ts.astype(jnp.int32)
                tf = trunc.astype(jnp.float32)
                ti = jnp.where(ts < tf, trunc - 1, trunc)
                dts = ts - ti.astype(jnp.float32)
                pol = jnp.where(p == 0.0, jnp.float32(-1.0), p)
                vl = pol * (jnp.float32(1.0) - dts)
                vr = pol * dts
                sidx = x.astype(jnp.int32) + y.astype(jnp.int32) * W
                sidx = jnp.clip(sidx, 0, HWSZ - 1)
                ok = ti >= 0
                m1 = ok & (ti < NBINS)
                m2 = ok & (ti < NBINS - 1)
                ti1 = jnp.clip(ti, 0, NBINS - 1)
                ti2 = jnp.clip(ti + 1, 0, NBINS - 1)
                i1 = sidx + ti1 * HWSZ
                i2 = sidx + ti2 * HWSZ
                v1 = jnp.where(m1, vl, jnp.float32(0.0))
                v2 = jnp.where(m2, vr, jnp.float32(0.0))
                r = v >> 2
                col = (v & 3) * 32
                idx_v[r, pl.ds(col, 16)] = i1
                idx_v[r, pl.ds(col + 16, 16)] = i2
                val_v[r, pl.ds(col, 16)] = v1
                val_v[r, pl.ds(col + 16, 16)] = v2

            def scat(j, cc):
                pltpu.sync_copy(val_v.at[j], grid.at[idx_v.at[j]], add=True)
                return cc

            bbase = b * (N * 4)

            def do_chunk(k, cc):
                start = bbase + (k * NSUB + s) * (CHUNK * 4)
                pltpu.sync_copy(ev_hbm.at[pl.ds(start, CHUNK * 4)], ev_v)

                def cv(v, c2):
                    compute_vec(v)
                    return c2
                lax.fori_loop(0, VECS, cv, 0)
                lax.fori_loop(0, ROWS, scat, 0)
                return cc

            ntrips = jnp.where(s == 0, NCHUNKS // NSUB + 1, NCHUNKS // NSUB)
            lax.fori_loop(0, ntrips, do_chunk, 0)

            plsc.subcore_barrier()
            pltpu.sync_copy(grid.at[pl.ds(s * STRIPE, STRIPE)],
                            out_hbm.at[b, pl.ds(s * STRIPE, STRIPE)])
            plsc.subcore_barrier()
            return carry

        lax.fori_loop(0, BATCH_PER_CORE, batch_body, 0)

    return voxel_sc


_voxel_sc = _make_sc_kernel()


@jax.jit
def _run(events_list):
    stamps = jnp.stack(
        [events_list[:, 0, 0], events_list[:, N - 1, 0]], axis=1)
    out = _voxel_sc(events_list.reshape(B * N * 4), stamps.reshape(16))
    return out.reshape(B, NBINS, H, W)


def kernel(events_list, device):
    return _run(events_list)


# speedup vs baseline: 6.8782x; 6.8782x over previous
import functools

import jax
import jax.numpy as jnp
from jax import lax
from jax.experimental import pallas as pl
from jax.experimental.pallas import tpu as pltpu
from jax.experimental.pallas import tpu_sc as plsc

NBINS = 5
H = 480
W = 640
B = 8
N = 1_000_000
HWSZ = H * W
G = NBINS * HWSZ

NSUB = 16
NCORE = 2
BATCH_PER_CORE = B // NCORE
CHUNK = 1600
EVN = CHUNK * 4
NCHUNKS = N // CHUNK
VECS = CHUNK // 16
ROWS = 2 * CHUNK // 128
STRIPE = G // NSUB
ZCHUNK = 6_000


def _make_sc_kernel():
    mesh = plsc.VectorSubcoreMesh(core_axis_name="c", subcore_axis_name="s")

    @functools.partial(
        pl.kernel,
        out_type=jax.ShapeDtypeStruct((B, G), jnp.float32),
        mesh=mesh,
        scratch_types=[
            pltpu.VMEM((2 * EVN,), jnp.float32),
            pltpu.VMEM((2 * ROWS, 128), jnp.int32),
            pltpu.VMEM((2 * ROWS, 128), jnp.float32),
            pltpu.VMEM((ZCHUNK,), jnp.float32),
            pltpu.VMEM((16,), jnp.float32),
            pltpu.VMEM_SHARED((G,), jnp.float32),
            pltpu.SemaphoreType.DMA,
            pltpu.SemaphoreType.DMA,
        ],
        compiler_params=pltpu.CompilerParams(needs_layout_passes=False),
    )
    def voxel_sc(ev_hbm, stamps_hbm, out_hbm, ev_v, idx_v, val_v, zbuf,
                 stamps_v, grid, ev_sem, sc_sem):
        c = lax.axis_index("c")
        s = lax.axis_index("s")

        def zb(i, carry):
            zbuf[pl.ds(i * 16, 16)] = jnp.zeros((16,), jnp.float32)
            return carry
        lax.fori_loop(0, ZCHUNK // 16, zb, 0)
        pltpu.sync_copy(stamps_hbm, stamps_v)

        lanes = lax.broadcasted_iota(jnp.int32, (16,), 0)

        def batch_body(bi, carry):
            b = c * BATCH_PER_CORE + bi
            def zg(j, cc):
                pltpu.sync_copy(
                    zbuf, grid.at[pl.ds(s * STRIPE + j * ZCHUNK, ZCHUNK)])
                return cc
            lax.fori_loop(0, STRIPE // ZCHUNK, zg, 0)
            plsc.subcore_barrier()

            b2 = lanes * 0 + b * 2
            first = plsc.load_gather(stamps_v, [b2])
            last = plsc.load_gather(stamps_v, [b2 + 1])
            d0 = last - first
            dT = jnp.where(d0 == 0.0, jnp.float32(1.0), d0)
            recip = jnp.float32(1.0) / dT
            first4 = first * jnp.float32(4.0)

            ntrips = jnp.where(s == 0, NCHUNKS // NSUB + 1, NCHUNKS // NSUB)

            def ev_copy(k, p):
                start = (k * NSUB + s) * EVN
                return pltpu.make_async_copy(
                    ev_hbm.at[b, pl.ds(start, EVN)],
                    ev_v.at[pl.ds(p * EVN, EVN)], ev_sem)

            def compute_vec(v, eoff, ioff):
                rows4 = eoff + v * 64 + lanes * 4
                t = plsc.load_gather(ev_v, [rows4])
                x = plsc.load_gather(ev_v, [rows4 + 1])
                y = plsc.load_gather(ev_v, [rows4 + 2])
                p = plsc.load_gather(ev_v, [rows4 + 3])
                ts = (t * jnp.float32(4.0) - first4) * recip
                trunc = ts.astype(jnp.int32)
                tf = trunc.astype(jnp.float32)
                ti = jnp.where(ts < tf, trunc - 1, trunc)
                dts = ts - ti.astype(jnp.float32)
                pol = jnp.where(p == 0.0, jnp.float32(-1.0), p)
                vl = pol * (jnp.float32(1.0) - dts)
                vr = pol * dts
                sidx = x.astype(jnp.int32) + y.astype(jnp.int32) * W
                ok = ti >= 0
                m1 = ok & (ti < NBINS)
                m2 = ok & (ti < NBINS - 1)
                ti1 = jnp.clip(ti, 0, NBINS - 1)
                ti2 = jnp.clip(ti + 1, 0, NBINS - 1)
                i1 = sidx + ti1 * HWSZ
                i2 = sidx + ti2 * HWSZ
                v1 = jnp.where(m1, vl, jnp.float32(0.0))
                v2 = jnp.where(m2, vr, jnp.float32(0.0))
                r = ioff + (v >> 2)
                col = (v & 3) * 32
                idx_v[r, pl.ds(col, 16)] = i1
                idx_v[r, pl.ds(col + 16, 16)] = i2
                val_v[r, pl.ds(col, 16)] = v1
                val_v[r, pl.ds(col + 16, 16)] = v2

            def fire(j, ioff):
                pltpu.async_copy(
                    val_v.at[ioff + j], grid.at[idx_v.at[ioff + j]],
                    sc_sem, add=True)

            def drain(j, ioff):
                pltpu.make_async_copy(
                    val_v.at[ioff + j], grid.at[idx_v.at[ioff + j]],
                    sc_sem).wait()

            ev_copy(0, 0).start()

            def do_chunk(k, cc):
                p = k & 1
                eoff = p * EVN
                ioff = p * ROWS
                ev_copy(k, p).wait()

                @pl.when(k + 1 < ntrips)
                def _():
                    ev_copy(k + 1, 1 - p).start()

                def cv(v, c2):
                    compute_vec(v, eoff, ioff)
                    return c2
                lax.fori_loop(0, VECS, cv, 0)

                @pl.when(k >= 1)
                def _():
                    def dr(j, c3):
                        drain(j, (1 - p) * ROWS)
                        return c3
                    lax.fori_loop(0, ROWS, dr, 0)

                def fi(j, c3):
                    fire(j, ioff)
                    return c3
                lax.fori_loop(0, ROWS, fi, 0)
                return cc

            lax.fori_loop(0, ntrips, do_chunk, 0)

            last_ioff = ((ntrips - 1) & 1) * ROWS

            def drl(j, cc):
                drain(j, last_ioff)
                return cc
            lax.fori_loop(0, ROWS, drl, 0)

            plsc.subcore_barrier()
            pltpu.sync_copy(grid.at[pl.ds(s * STRIPE, STRIPE)],
                            out_hbm.at[b, pl.ds(s * STRIPE, STRIPE)])
            plsc.subcore_barrier()
            return carry

        lax.fori_loop(0, BATCH_PER_CORE, batch_body, 0)

    return voxel_sc


_voxel_sc = _make_sc_kernel()


@jax.jit
def _run(events_list):
    stamps = jnp.stack(
        [events_list[:, 0, 0], events_list[:, N - 1, 0]], axis=1)
    out = _voxel_sc(events_list.reshape(B, N * 4), stamps.reshape(16))
    return out.reshape(B, NBINS, H, W)


def kernel(events_list, device):
    return _run(events_list)
